# Initial kernel scaffold; baseline (speedup 1.0000x reference)
import jax
import jax.numpy as jnp
from jax.experimental import pallas as pl


def kernel(x, edge_index, W_lin0, b_lin0, W_upd0, b_upd0, W_lin1, b_lin1, W_upd1, b_upd1, W_lin2, b_lin2, W_upd2, b_upd2, W_pred, b_pred):
    raise NotImplementedError("write your pallas kernel here")


# speedup vs baseline: 4.1583x; 4.1583x over previous
import functools

import jax
import jax.numpy as jnp
from jax import lax
from jax.experimental import pallas as pl
from jax.experimental.pallas import tpu as pltpu
from jax.experimental.pallas import tpu_sc as plsc

N = 10000
E = 320000
D = 128
DL = 16
NPAD = 10240
TRASH = N
NC = 2
NS = 16
NW = NC * NS
EW = E // NW
B = 80
NCH = EW // B
KPC = B // 16
CR = NPAD // D
RO = NPAD // NS
RH = 32
NRH = RO // RH


def _sc_body(table, src_hbm, dst_hbm, flag_hbm, g_out, cnt_out, srcb, dstb,
             rows, obuf, cbuf, fbuf, idx2, gsh, sem):
    c = lax.axis_index("c")
    s = lax.axis_index("s")
    wid = c * NS + s
    ebase = wid * EW

    iota16 = jax.lax.iota(jnp.int32, 16)

    def idfill(q, carry):
        base = s * RO + q * RH
        idx2[q, pl.ds(0, 16)] = base + iota16
        idx2[q, pl.ds(16, 16)] = base + 16 + iota16
        return carry

    lax.fori_loop(0, NRH, idfill, 0)

    pltpu.sync_copy(flag_hbm, fbuf)
    fl = fbuf[...][0]

    zero16 = jnp.zeros((16,), jnp.float32)
    one16 = jnp.ones((16,), jnp.float32)

    def zrow(r, carry):
        for kk in range(D // 16):
            obuf[r, pl.ds(kk * 16, 16)] = zero16
        return carry

    lax.fori_loop(0, RH, zrow, 0)

    def zcopy(q, carry):
        pltpu.sync_copy(obuf, gsh.at[idx2.at[q]])
        return carry

    lax.fori_loop(0, NRH, zcopy, 0)

    plsc.subcore_barrier()

    trash16 = jnp.full((16,), TRASH, jnp.int32)

    @pl.when(fl != 0)
    def _():
        def orow(r, carry):
            for kk in range(D // 16):
                rows[r, pl.ds(kk * 16, 16)] = one16
            return carry

        lax.fori_loop(0, B, orow, 0)

    def mainloop(j, carry):
        pltpu.sync_copy(src_hbm.at[pl.ds(ebase + j * B, B)], srcb.at[0])
        pltpu.sync_copy(dst_hbm.at[pl.ds(ebase + j * B, B)], dstb.at[0])
        for k in range(KPC):
            sv = srcb[0, pl.ds(k * 16, 16)]
            dv = dstb[0, pl.ds(k * 16, 16)]
            dstb[0, pl.ds(k * 16, 16)] = jnp.where(sv == dv, trash16, dv)

        @pl.when(fl == 0)
        def _():
            pltpu.async_copy(table.at[srcb.at[0]], rows, sem).wait()

        pltpu.sync_copy(rows, gsh.at[dstb.at[0]], add=True)
        return carry

    lax.fori_loop(0, NCH, mainloop, 0)

    plsc.subcore_barrier()

    @pl.when(fl == 0)
    def _():
        def ocopy(q, carry):
            pltpu.async_copy(gsh.at[idx2.at[q]], obuf, sem).wait()
            pltpu.sync_copy(obuf, g_out.at[c, pl.ds(s * RO + q * RH, RH)])
            return carry

        lax.fori_loop(0, NRH, ocopy, 0)

    @pl.when(fl != 0)
    def _():
        def ccopy(q, carry):
            pltpu.async_copy(gsh.at[idx2.at[q]], obuf, sem).wait()

            def crow(r, carry2):
                cbuf[r, pl.ds(0, 16)] = obuf[r, pl.ds(0, 16)]
                return carry2

            lax.fori_loop(0, RH, crow, 0)
            pltpu.sync_copy(cbuf, cnt_out.at[c, pl.ds(s * RO + q * RH, RH)])
            return carry

        lax.fori_loop(0, NRH, ccopy, 0)


def _make_sc_pass():
    mesh = plsc.VectorSubcoreMesh(
        core_axis_name="c", subcore_axis_name="s",
        num_cores=NC, num_subcores=NS)
    out_type = [jax.ShapeDtypeStruct((NC, NPAD, D), jnp.float32),
                jax.ShapeDtypeStruct((NC, NPAD, 16), jnp.float32)]
    scratch = [
        pltpu.VMEM((1, B), jnp.int32),
        pltpu.VMEM((1, B), jnp.int32),
        pltpu.VMEM((B, D), jnp.float32),
        pltpu.VMEM((RH, D), jnp.float32),
        pltpu.VMEM((RH, 16), jnp.float32),
        pltpu.VMEM((16,), jnp.int32),
        pltpu.VMEM((NRH, RH), jnp.int32),
        pltpu.VMEM_SHARED((NPAD, D), jnp.float32),
        pltpu.SemaphoreType.DMA,
    ]
    return pl.kernel(_sc_body, out_type=out_type, mesh=mesh,
                     scratch_types=scratch)


_sc_pass_cache = functools.cache(_make_sc_pass)


def _tc_body(*refs, relu, project):
    if project:
        g_ref, cnt_ref, h_ref, wl, bl, wu, bu, wp, bp, o_ref = refs
    else:
        g_ref, cnt_ref, h_ref, wl, bl, wu, bu, o_ref = refs
    g = g_ref[0] + g_ref[1]
    mm = functools.partial(jnp.dot, preferred_element_type=jnp.float32)
    aggr = mm(g, wl[...]) + cnt_ref[...] * bl[...]
    out = mm(aggr, wu[:D]) + mm(h_ref[...], wu[D:]) + bu[...]
    if relu:
        out = jnp.maximum(out, 0.0)
    if project:
        out = mm(out, wp[...]) + bp[...]
    o_ref[...] = out


BM = 1000


def _make_tc_layer(relu, project):
    dout = DL if project else D
    in_specs = [
        pl.BlockSpec((NC, BM, D), lambda m: (0, m, 0)),
        pl.BlockSpec((BM, 1), lambda m: (m, 0)),
        pl.BlockSpec((BM, D), lambda m: (m, 0)),
        pl.BlockSpec((D, D), lambda m: (0, 0)),
        pl.BlockSpec((1, D), lambda m: (0, 0)),
        pl.BlockSpec((2 * D, D), lambda m: (0, 0)),
        pl.BlockSpec((1, D), lambda m: (0, 0)),
    ]
    if project:
        in_specs += [
            pl.BlockSpec((D, DL), lambda m: (0, 0)),
            pl.BlockSpec((1, DL), lambda m: (0, 0)),
        ]
    return pl.pallas_call(
        functools.partial(_tc_body, relu=relu, project=project),
        grid=(N // BM,),
        in_specs=in_specs,
        out_specs=pl.BlockSpec((BM, dout), lambda m: (m, 0)),
        out_shape=jax.ShapeDtypeStruct((N, dout), jnp.float32),
    )


_tc_hidden = _make_tc_layer(relu=True, project=False)
_tc_final = _make_tc_layer(relu=False, project=True)


def kernel(x, edge_index,
           W_lin0, b_lin0, W_upd0, b_upd0,
           W_lin1, b_lin1, W_upd1, b_upd1,
           W_lin2, b_lin2, W_upd2, b_upd2,
           W_pred, b_pred):
    r = lambda b: b.reshape(1, -1)
    sc_pass = _sc_pass_cache()
    srcs = edge_index[0]
    dsts = edge_index[1]
    fl0 = jnp.zeros((16,), jnp.int32)
    fl1 = jnp.ones((16,), jnp.int32)
    _, cnt_parts = sc_pass(x, srcs, dsts, fl1)
    cnt = (cnt_parts[0] + cnt_parts[1])[:N, :1]
    g0, _ = sc_pass(x, srcs, dsts, fl0)
    h1 = _tc_hidden(g0, cnt, x, W_lin0, r(b_lin0), W_upd0, r(b_upd0))
    g1, _ = sc_pass(h1, srcs, dsts, fl0)
    h2 = _tc_hidden(g1, cnt, h1, W_lin1, r(b_lin1), W_upd1, r(b_upd1))
    g2, _ = sc_pass(h2, srcs, dsts, fl0)
    y = _tc_final(g2, cnt, h2, W_lin2, r(b_lin2), W_upd2, r(b_upd2),
                  W_pred, r(b_pred))
    return y


# speedup vs baseline: 6.3094x; 1.5173x over previous
import functools

import jax
import jax.numpy as jnp
from jax import lax
from jax.experimental import pallas as pl
from jax.experimental.pallas import tpu as pltpu
from jax.experimental.pallas import tpu_sc as plsc

N = 10000
E = 320000
D = 128
DL = 16
NPAD = 10240
TRASH = N
NC = 2
NS = 16
NW = NC * NS
EW = E // NW
B = 128
NCH = EW // B
NPAIR = NCH // 2
KPC = B // 16
TAIL = EW - NCH * B
TOFF = NCH * B
CR = NPAD // D
RO = NPAD // NS
RH = 32
NRH = RO // RH


def _sc_body(table, src_hbm, dst_hbm, flag_hbm, g_out, cnt_out, srcb, dstb,
             rows0, rows1, obuf, cbuf, fbuf, tsrc, tdst, idx2, gsh,
             semg0, semg1, sema0, sema1, sem):
    c = lax.axis_index("c")
    s = lax.axis_index("s")
    wid = c * NS + s
    ebase = wid * EW

    iota16 = jax.lax.iota(jnp.int32, 16)

    def idfill(q, carry):
        base = s * RO + q * RH
        idx2[q, pl.ds(0, 16)] = base + iota16
        idx2[q, pl.ds(16, 16)] = base + 16 + iota16
        return carry

    lax.fori_loop(0, NRH, idfill, 0)

    pltpu.sync_copy(flag_hbm, fbuf)
    fl = fbuf[...][0]

    zero16 = jnp.zeros((16,), jnp.float32)
    one16 = jnp.ones((16,), jnp.float32)

    def zrow(r, carry):
        for kk in range(D // 16):
            obuf[r, pl.ds(kk * 16, 16)] = zero16
        return carry

    lax.fori_loop(0, RH, zrow, 0)

    def zcopy(q, carry):
        pltpu.sync_copy(obuf, gsh.at[idx2.at[q]])
        return carry

    lax.fori_loop(0, NRH, zcopy, 0)

    plsc.subcore_barrier()

    trash16 = jnp.full((16,), TRASH, jnp.int32)

    def load_fix(j, slot):
        pltpu.sync_copy(src_hbm.at[pl.ds(ebase + j * B, B)], srcb.at[slot])
        pltpu.sync_copy(dst_hbm.at[pl.ds(ebase + j * B, B)], dstb.at[slot])
        for k in range(KPC):
            sv = srcb[slot, pl.ds(k * 16, 16)]
            dv = dstb[slot, pl.ds(k * 16, 16)]
            dstb[slot, pl.ds(k * 16, 16)] = jnp.where(sv == dv, trash16, dv)

    def load_fix_tail():
        pltpu.sync_copy(src_hbm.at[pl.ds(ebase + TOFF, TAIL)], tsrc.at[0])
        pltpu.sync_copy(dst_hbm.at[pl.ds(ebase + TOFF, TAIL)], tdst.at[0])
        sv = tsrc[0, pl.ds(0, 16)]
        dv = tdst[0, pl.ds(0, 16)]
        tdst[0, pl.ds(0, 16)] = jnp.where(sv == dv, trash16, dv)

    @pl.when(fl == 0)
    def _():
        def pair(q, carry):
            j0 = 2 * q
            load_fix(j0, 0)
            ga = pltpu.async_copy(table.at[srcb.at[0]], rows0, semg0)
            load_fix(j0 + 1, 1)
            ga.wait()
            sa = pltpu.async_copy(rows0, gsh.at[dstb.at[0]], sema0, add=True)
            gb = pltpu.async_copy(table.at[srcb.at[1]], rows1, semg1)
            gb.wait()
            sa.wait()
            sb = pltpu.async_copy(rows1, gsh.at[dstb.at[1]], sema1, add=True)
            sb.wait()
            return carry

        lax.fori_loop(0, NPAIR, pair, 0)
        load_fix_tail()
        pltpu.async_copy(
            table.at[tsrc.at[0]], rows0.at[pl.ds(0, TAIL)], semg0).wait()
        pltpu.sync_copy(rows0.at[pl.ds(0, TAIL)], gsh.at[tdst.at[0]],
                        add=True)

    @pl.when(fl != 0)
    def _():
        def orow(r, carry):
            for kk in range(D // 16):
                rows0[r, pl.ds(kk * 16, 16)] = one16
            return carry

        lax.fori_loop(0, B, orow, 0)

        def cntloop(j, carry):
            load_fix(j, 0)
            pltpu.sync_copy(rows0, gsh.at[dstb.at[0]], add=True)
            return carry

        lax.fori_loop(0, NCH, cntloop, 0)
        load_fix_tail()
        pltpu.sync_copy(rows0.at[pl.ds(0, TAIL)], gsh.at[tdst.at[0]],
                        add=True)

    plsc.subcore_barrier()

    @pl.when(fl == 0)
    def _():
        def ocopy(q, carry):
            pltpu.async_copy(gsh.at[idx2.at[q]], obuf, sem).wait()
            pltpu.sync_copy(obuf, g_out.at[c, pl.ds(s * RO + q * RH, RH)])
            return carry

        lax.fori_loop(0, NRH, ocopy, 0)

    @pl.when(fl != 0)
    def _():
        def ccopy(q, carry):
            pltpu.async_copy(gsh.at[idx2.at[q]], obuf, sem).wait()

            def crow(r, carry2):
                cbuf[r, pl.ds(0, 16)] = obuf[r, pl.ds(0, 16)]
                return carry2

            lax.fori_loop(0, RH, crow, 0)
            pltpu.sync_copy(cbuf, cnt_out.at[c, pl.ds(s * RO + q * RH, RH)])
            return carry

        lax.fori_loop(0, NRH, ccopy, 0)


def _make_sc_pass():
    mesh = plsc.VectorSubcoreMesh(
        core_axis_name="c", subcore_axis_name="s",
        num_cores=NC, num_subcores=NS)
    out_type = [jax.ShapeDtypeStruct((NC, NPAD, D), jnp.float32),
                jax.ShapeDtypeStruct((NC, NPAD, 16), jnp.float32)]
    scratch = [
        pltpu.VMEM((2, B), jnp.int32),
        pltpu.VMEM((2, B), jnp.int32),
        pltpu.VMEM((B, D), jnp.float32),
        pltpu.VMEM((B, D), jnp.float32),
        pltpu.VMEM((RH, D), jnp.float32),
        pltpu.VMEM((RH, 16), jnp.float32),
        pltpu.VMEM((16,), jnp.int32),
        pltpu.VMEM((1, 16), jnp.int32),
        pltpu.VMEM((1, 16), jnp.int32),
        pltpu.VMEM((NRH, RH), jnp.int32),
        pltpu.VMEM_SHARED((NPAD, D), jnp.float32),
        pltpu.SemaphoreType.DMA,
        pltpu.SemaphoreType.DMA,
        pltpu.SemaphoreType.DMA,
        pltpu.SemaphoreType.DMA,
        pltpu.SemaphoreType.DMA,
    ]
    return pl.kernel(_sc_body, out_type=out_type, mesh=mesh,
                     scratch_types=scratch)


_sc_pass_cache = functools.cache(_make_sc_pass)


def _tc_body(*refs, relu, project):
    if project:
        g_ref, cnt_ref, h_ref, wl, bl, wu, bu, wp, bp, o_ref = refs
    else:
        g_ref, cnt_ref, h_ref, wl, bl, wu, bu, o_ref = refs
    g = g_ref[0] + g_ref[1]
    mm = functools.partial(jnp.dot, preferred_element_type=jnp.float32)
    aggr = mm(g, wl[...]) + cnt_ref[...] * bl[...]
    out = mm(aggr, wu[:D]) + mm(h_ref[...], wu[D:]) + bu[...]
    if relu:
        out = jnp.maximum(out, 0.0)
    if project:
        out = mm(out, wp[...]) + bp[...]
    o_ref[...] = out


BM = 1000


def _make_tc_layer(relu, project):
    dout = DL if project else D
    in_specs = [
        pl.BlockSpec((NC, BM, D), lambda m: (0, m, 0)),
        pl.BlockSpec((BM, 1), lambda m: (m, 0)),
        pl.BlockSpec((BM, D), lambda m: (m, 0)),
        pl.BlockSpec((D, D), lambda m: (0, 0)),
        pl.BlockSpec((1, D), lambda m: (0, 0)),
        pl.BlockSpec((2 * D, D), lambda m: (0, 0)),
        pl.BlockSpec((1, D), lambda m: (0, 0)),
    ]
    if project:
        in_specs += [
            pl.BlockSpec((D, DL), lambda m: (0, 0)),
            pl.BlockSpec((1, DL), lambda m: (0, 0)),
        ]
    return pl.pallas_call(
        functools.partial(_tc_body, relu=relu, project=project),
        grid=(N // BM,),
        in_specs=in_specs,
        out_specs=pl.BlockSpec((BM, dout), lambda m: (m, 0)),
        out_shape=jax.ShapeDtypeStruct((N, dout), jnp.float32),
    )


_tc_hidden = _make_tc_layer(relu=True, project=False)
_tc_final = _make_tc_layer(relu=False, project=True)


def kernel(x, edge_index,
           W_lin0, b_lin0, W_upd0, b_upd0,
           W_lin1, b_lin1, W_upd1, b_upd1,
           W_lin2, b_lin2, W_upd2, b_upd2,
           W_pred, b_pred):
    r = lambda b: b.reshape(1, -1)
    sc_pass = _sc_pass_cache()
    srcs = edge_index[0]
    dsts = edge_index[1]
    fl0 = jnp.zeros((16,), jnp.int32)
    fl1 = jnp.ones((16,), jnp.int32)
    _, cnt_parts = sc_pass(x, srcs, dsts, fl1)
    cnt = (cnt_parts[0] + cnt_parts[1])[:N, :1]
    g0, _ = sc_pass(x, srcs, dsts, fl0)
    h1 = _tc_hidden(g0, cnt, x, W_lin0, r(b_lin0), W_upd0, r(b_upd0))
    g1, _ = sc_pass(h1, srcs, dsts, fl0)
    h2 = _tc_hidden(g1, cnt, h1, W_lin1, r(b_lin1), W_upd1, r(b_upd1))
    g2, _ = sc_pass(h2, srcs, dsts, fl0)
    y = _tc_final(g2, cnt, h2, W_lin2, r(b_lin2), W_upd2, r(b_upd2),
                  W_pred, r(b_pred))
    return y


# speedup vs baseline: 7.7728x; 1.2319x over previous
import functools

import jax
import jax.numpy as jnp
from jax import lax
from jax.experimental import pallas as pl
from jax.experimental.pallas import tpu as pltpu
from jax.experimental.pallas import tpu_sc as plsc

N = 10000
E = 320000
D = 128
DL = 16
NPAD = 10240
TRASH = N
NC = 2
NS = 16
NW = NC * NS
EW = E // NW
B = 80
NCH = EW // B
NPAIR = NCH // 2
KPC = B // 16
IB = 25
RO = NPAD // NS
RH = 80
NRH = RO // RH


def _sc_body(table, src_hbm, dst_hbm, flag_hbm, g_out, src_all, dst_all,
             rows0, rows1, fbuf, idbuf, gsh, semi, semg0, semg1, sema0,
             sema1, sem):
    c = lax.axis_index("c")
    s = lax.axis_index("s")
    wid = c * NS + s
    ebase = wid * EW

    iota16 = jax.lax.iota(jnp.int32, 16)
    zero16 = jnp.zeros((16,), jnp.float32)
    one16 = jnp.ones((16,), jnp.float32)
    trash16 = jnp.full((16,), TRASH, jnp.int32)

    pltpu.sync_copy(flag_hbm, fbuf)
    fl = fbuf[...][0]

    def idfill(q):
        base = s * RO + q * RH
        for kk in range(KPC):
            idbuf[0, pl.ds(kk * 16, 16)] = base + kk * 16 + iota16

    def zrow(r, carry):
        for kk in range(D // 16):
            rows0[r, pl.ds(kk * 16, 16)] = zero16
        return carry

    lax.fori_loop(0, RH, zrow, 0)

    def zcopy(q, carry):
        idfill(q)
        pltpu.sync_copy(rows0, gsh.at[idbuf.at[0]])
        return carry

    lax.fori_loop(0, NRH, zcopy, 0)

    plsc.subcore_barrier()

    pltpu.sync_copy(src_hbm.at[pl.ds(ebase, EW)], src_all)

    for bb in range(NCH // IB):
        def istart(j, carry):
            pltpu.async_copy(
                dst_hbm.at[pl.ds(ebase + j * B, B)], dst_all.at[j], semi)
            return carry

        def iwait(j, carry):
            pltpu.make_async_copy(
                dst_hbm.at[pl.ds(ebase + j * B, B)], dst_all.at[j],
                semi).wait()
            return carry

        lax.fori_loop(bb * IB, (bb + 1) * IB, istart, 0)
        lax.fori_loop(bb * IB, (bb + 1) * IB, iwait, 0)

    def fixup(j, carry):
        for k in range(KPC):
            sv = src_all[pl.ds(j * B + k * 16, 16)]
            dv = dst_all[j, pl.ds(k * 16, 16)]
            dst_all[j, pl.ds(k * 16, 16)] = jnp.where(sv == dv, trash16, dv)
        return carry

    lax.fori_loop(0, NCH, fixup, 0)

    def g_start(j, rows, sg):
        pltpu.async_copy(table.at[src_all.at[pl.ds(j * B, B)]], rows, sg)

    def g_wait(j, rows, sg):
        pltpu.make_async_copy(
            table.at[src_all.at[pl.ds(j * B, B)]], rows, sg).wait()

    def s_start(j, rows, sa):
        pltpu.async_copy(rows, gsh.at[dst_all.at[j]], sa, add=True)

    def s_wait(j, rows, sa):
        pltpu.make_async_copy(rows, gsh.at[dst_all.at[j]], sa).wait()

    @pl.when(fl == 0)
    def _():
        g_start(0, rows0, semg0)

        def pair(q, carry):
            j0 = 2 * q
            g_wait(j0, rows0, semg0)
            s_start(j0, rows0, sema0)

            @pl.when(q > 0)
            def _():
                s_wait(j0 - 1, rows1, sema1)

            g_start(j0 + 1, rows1, semg1)
            g_wait(j0 + 1, rows1, semg1)
            s_start(j0 + 1, rows1, sema1)
            s_wait(j0, rows0, sema0)
            g_start(j0 + 2, rows0, semg0)
            return carry

        lax.fori_loop(0, NPAIR, pair, 0)
        s_wait(NCH - 2, rows1, sema1)
        g_wait(NCH - 1, rows0, semg0)
        s_start(NCH - 1, rows0, sema0)
        s_wait(NCH - 1, rows0, sema0)

    @pl.when(fl != 0)
    def _():
        def orow(r, carry):
            for kk in range(D // 16):
                rows0[r, pl.ds(kk * 16, 16)] = one16
                rows1[r, pl.ds(kk * 16, 16)] = one16
            return carry

        lax.fori_loop(0, B, orow, 0)

        def cpair(q, carry):
            j0 = 2 * q
            s_start(j0, rows0, sema0)

            @pl.when(q > 0)
            def _():
                s_wait(j0 - 1, rows1, sema1)

            s_start(j0 + 1, rows1, sema1)
            s_wait(j0, rows0, sema0)
            return carry

        lax.fori_loop(0, NPAIR, cpair, 0)
        s_wait(NCH - 2, rows1, sema1)
        s_start(NCH - 1, rows0, sema0)
        s_wait(NCH - 1, rows0, sema0)

    plsc.subcore_barrier()

    def ocopy(q, carry):
        idfill(q)
        pltpu.async_copy(gsh.at[idbuf.at[0]], rows0, sem).wait()
        pltpu.sync_copy(rows0, g_out.at[c, pl.ds(s * RO + q * RH, RH)])
        return carry

    lax.fori_loop(0, NRH, ocopy, 0)


def _make_sc_pass():
    mesh = plsc.VectorSubcoreMesh(
        core_axis_name="c", subcore_axis_name="s",
        num_cores=NC, num_subcores=NS)
    out_type = [jax.ShapeDtypeStruct((NC, NPAD, D), jnp.float32)]
    scratch = [
        pltpu.VMEM((EW,), jnp.int32),
        pltpu.VMEM((NCH, B), jnp.int32),
        pltpu.VMEM((B, D), jnp.float32),
        pltpu.VMEM((B, D), jnp.float32),
        pltpu.VMEM((16,), jnp.int32),
        pltpu.VMEM((1, B), jnp.int32),
        pltpu.VMEM_SHARED((NPAD, D), jnp.float32),
        pltpu.SemaphoreType.DMA,
        pltpu.SemaphoreType.DMA,
        pltpu.SemaphoreType.DMA,
        pltpu.SemaphoreType.DMA,
        pltpu.SemaphoreType.DMA,
        pltpu.SemaphoreType.DMA,
    ]
    return pl.kernel(_sc_body, out_type=out_type, mesh=mesh,
                     scratch_types=scratch)


_sc_pass_cache = functools.cache(_make_sc_pass)


def _tc_body(*refs, relu, project):
    if project:
        g_ref, cnt_ref, h_ref, wl, bl, wu, bu, wp, bp, o_ref = refs
    else:
        g_ref, cnt_ref, h_ref, wl, bl, wu, bu, o_ref = refs
    g = g_ref[0] + g_ref[1]
    mm = functools.partial(jnp.dot, preferred_element_type=jnp.float32)
    aggr = mm(g, wl[...]) + cnt_ref[...] * bl[...]
    out = mm(aggr, wu[:D]) + mm(h_ref[...], wu[D:]) + bu[...]
    if relu:
        out = jnp.maximum(out, 0.0)
    if project:
        out = mm(out, wp[...]) + bp[...]
    o_ref[...] = out


BM = 1000


def _make_tc_layer(relu, project):
    dout = DL if project else D
    in_specs = [
        pl.BlockSpec((NC, BM, D), lambda m: (0, m, 0)),
        pl.BlockSpec((BM, 1), lambda m: (m, 0)),
        pl.BlockSpec((BM, D), lambda m: (m, 0)),
        pl.BlockSpec((D, D), lambda m: (0, 0)),
        pl.BlockSpec((1, D), lambda m: (0, 0)),
        pl.BlockSpec((2 * D, D), lambda m: (0, 0)),
        pl.BlockSpec((1, D), lambda m: (0, 0)),
    ]
    if project:
        in_specs += [
            pl.BlockSpec((D, DL), lambda m: (0, 0)),
            pl.BlockSpec((1, DL), lambda m: (0, 0)),
        ]
    return pl.pallas_call(
        functools.partial(_tc_body, relu=relu, project=project),
        grid=(N // BM,),
        in_specs=in_specs,
        out_specs=pl.BlockSpec((BM, dout), lambda m: (m, 0)),
        out_shape=jax.ShapeDtypeStruct((N, dout), jnp.float32),
    )


_tc_hidden = _make_tc_layer(relu=True, project=False)
_tc_final = _make_tc_layer(relu=False, project=True)


def kernel(x, edge_index,
           W_lin0, b_lin0, W_upd0, b_upd0,
           W_lin1, b_lin1, W_upd1, b_upd1,
           W_lin2, b_lin2, W_upd2, b_upd2,
           W_pred, b_pred):
    r = lambda b: b.reshape(1, -1)
    sc_pass = _sc_pass_cache()
    srcs = edge_index[0]
    dsts = edge_index[1]
    fl0 = jnp.zeros((16,), jnp.int32)
    fl1 = jnp.ones((16,), jnp.int32)
    [cntg] = sc_pass(x, srcs, dsts, fl1)
    cnt = (cntg[0] + cntg[1])[:N, :1]
    [g0] = sc_pass(x, srcs, dsts, fl0)
    h1 = _tc_hidden(g0, cnt, x, W_lin0, r(b_lin0), W_upd0, r(b_upd0))
    [g1] = sc_pass(h1, srcs, dsts, fl0)
    h2 = _tc_hidden(g1, cnt, h1, W_lin1, r(b_lin1), W_upd1, r(b_upd1))
    [g2] = sc_pass(h2, srcs, dsts, fl0)
    y = _tc_final(g2, cnt, h2, W_lin2, r(b_lin2), W_upd2, r(b_upd2),
                  W_pred, r(b_pred))
    return y
